# baseline (speedup 1.0000x reference)
import functools

import jax
import jax.numpy as jnp
from jax import lax
from jax.experimental import pallas as pl
from jax.experimental.pallas import tpu as pltpu
from jax.experimental.pallas import tpu_sc as plsc

_B, _H, _N, _DH = 16, 12, 577, 64
_S = 255
_L = 256
_EPS = 1e-6
_NM1 = _N - 1


def _diag_to_row(col, n):
    bc = jnp.broadcast_to(col, (n, n))
    ii = lax.broadcasted_iota(jnp.int32, (n, n), 0)
    jj = lax.broadcasted_iota(jnp.int32, (n, n), 1)
    return jnp.sum(jnp.where(ii == jj, bc, jnp.zeros_like(bc)), axis=0,
                   keepdims=True)


def _diag_to_col(row, n):
    bc = jnp.broadcast_to(row, (n, n))
    ii = lax.broadcasted_iota(jnp.int32, (n, n), 0)
    jj = lax.broadcasted_iota(jnp.int32, (n, n), 1)
    return jnp.sum(jnp.where(ii == jj, bc, jnp.zeros_like(bc)), axis=1,
                   keepdims=True)


def _cumsum_col(a, n):
    k = 1
    while k < n:
        shifted = jnp.concatenate(
            [jnp.zeros((k, 1), a.dtype), a[: n - k, :]], axis=0)
        a = a + shifted
        k *= 2
    return a


def _part1_body(x_ref, clst_ref, maskt_ref, steps_ref, uid_ref, nm_ref):
    sig = jnp.zeros((_NM1, 1), jnp.float32)
    for h in range(_H):
        xh = x_ref[0, h]
        ssq = jnp.sum(xh * xh, axis=1, keepdims=True)
        norms = jnp.sqrt(ssq[1:, :])
        sig = sig + clst_ref[0, :, h : h + 1] * norms
    denom = jnp.sum(sig) + _EPS
    normed = sig / denom
    cdf = _cumsum_col(normed, _NM1)
    mrow = maskt_ref[0][:, 1:]
    mcol = _diag_to_col(mrow, _NM1)
    cdf = jnp.where(mcol > 0.5, cdf, cdf + 0.1)

    steps = steps_ref[...]
    dist = jnp.abs(steps - cdf)
    dmin = jnp.min(dist, axis=0, keepdims=True)
    nidx = lax.broadcasted_iota(jnp.int32, (_NM1, _S), 0)
    idx_row = jnp.min(jnp.where(dist == dmin, nidx, _NM1), axis=0,
                      keepdims=True)
    ids_row = idx_row + 1

    n = _S
    ii = lax.broadcasted_iota(jnp.int32, (n, n), 0)
    jj = lax.broadcasted_iota(jnp.int32, (n, n), 1)
    ids_col = _diag_to_col(ids_row, n)
    ids_row_bc = jnp.broadcast_to(ids_row, (n, n))
    ids_col_bc = jnp.broadcast_to(ids_col, (n, n))
    lt = (ids_row_bc < ids_col_bc).astype(jnp.int32)
    eq_before = ((ids_row_bc == ids_col_bc) & (jj < ii)).astype(jnp.int32)
    rank_col = jnp.sum(lt + eq_before, axis=1, keepdims=True)

    rank_row = _diag_to_row(rank_col, n)
    rank_row_bc = jnp.broadcast_to(rank_row, (n, n))
    onehot = rank_row_bc == ii
    s_col = jnp.sum(jnp.where(onehot, ids_row_bc, jnp.zeros_like(ids_row_bc)),
                    axis=1, keepdims=True)

    s_prev = jnp.concatenate(
        [jnp.full((1, 1), -1, jnp.int32), s_col[:-1, :]], axis=0)
    keep_col = (s_col != s_prev).astype(jnp.int32)
    keep_row = _diag_to_row(keep_col, n)
    keep_row_bc = jnp.broadcast_to(keep_row, (n, n))
    pos_col = jnp.sum(jnp.where(jj < ii, keep_row_bc, jnp.zeros_like(keep_row_bc)),
                      axis=1, keepdims=True)
    pos_col_bc = jnp.broadcast_to(pos_col, (n, n))
    s_col_bc = jnp.broadcast_to(s_col, (n, n))
    sel = (pos_col_bc == jj) & (keep_col != 0)
    uidv_row = jnp.sum(jnp.where(sel, s_col_bc, jnp.zeros_like(s_col_bc)),
                       axis=0, keepdims=True)

    uid_row = jnp.concatenate(
        [jnp.zeros((1, 1), jnp.int32), uidv_row], axis=1)
    pos0 = lax.broadcasted_iota(jnp.int32, (1, _L), 1)
    nm_row = ((uid_row != 0) | (pos0 == 0)).astype(jnp.int32)

    uid_ref[0] = uid_row
    nm_ref[0] = nm_row


def _part1(x, clst, maskt, steps_row, interpret=False):
    return pl.pallas_call(
        _part1_body,
        grid=(_B,),
        in_specs=[
            pl.BlockSpec((1, _H, _N, _DH), lambda b: (b, 0, 0, 0)),
            pl.BlockSpec((1, _NM1, _H), lambda b: (b, 0, 0)),
            pl.BlockSpec((1, 1, _N), lambda b: (b, 0, 0)),
            pl.BlockSpec((1, _S), lambda b: (0, 0)),
        ],
        out_specs=[
            pl.BlockSpec((1, 1, _L), lambda b: (b, 0, 0)),
            pl.BlockSpec((1, 1, _L), lambda b: (b, 0, 0)),
        ],
        out_shape=[
            jax.ShapeDtypeStruct((_B, 1, _L), jnp.int32),
            jax.ShapeDtypeStruct((_B, 1, _L), jnp.int32),
        ],
        interpret=interpret,
    )(x, clst, maskt.reshape(_B, 1, _N), steps_row)


_R = _B * _H * _L
_NW = 32
_RPW = _R // _NW
_CH = 128
_NCH = _RPW // _CH


def _sc_gather_body(table_hbm, uidflat_hbm, out_hbm, idx_v, rows_v, sem):
    wid = lax.axis_index("s") * 2 + lax.axis_index("c")
    base = wid * _RPW
    for t in range(_NCH):
        off = base + t * _CH
        pltpu.sync_copy(uidflat_hbm.at[pl.ds(off, _CH)], idx_v)
        pltpu.async_copy(table_hbm.at[idx_v], rows_v, sem).wait()
        pltpu.sync_copy(rows_v, out_hbm.at[pl.ds(off, _CH)])


def _make_sc_gather():
    mesh = plsc.VectorSubcoreMesh(core_axis_name="c", subcore_axis_name="s")
    return functools.partial(
        pl.kernel,
        mesh=mesh,
        out_type=jax.ShapeDtypeStruct((_R, _N), jnp.float32),
        scratch_types=[
            pltpu.VMEM((_CH,), jnp.int32),
            pltpu.VMEM((_CH, _N), jnp.float32),
            pltpu.SemaphoreType.DMA,
        ],
        compiler_params=pltpu.CompilerParams(use_tc_tiling_on_sc=False),
    )(_sc_gather_body)


def kernel(x, attn, mask, sample_steps):
    clst = jnp.transpose(attn[:, :, 0, 1:], (0, 2, 1))
    maskt = mask.astype(jnp.float32)
    steps_row = sample_steps.reshape(1, _S)
    uid, nm = _part1(x, clst, maskt, steps_row)
    uid = uid.reshape(_B, _L)
    new_mask = nm.reshape(_B, _L).astype(bool)

    boff = (jnp.arange(_B * _H, dtype=jnp.int32) * _N).reshape(_B, _H)
    gidx = (uid[:, None, :] + boff[:, :, None]).reshape(_R)

    _DEBUG_TC_GATHER = True
    if _DEBUG_TC_GATHER:
        exp = jnp.repeat(uid[:, None, :], _H, axis=1)
        idx = jnp.broadcast_to(exp[:, :, :, None], (_B, _H, _L, _N))
        new_attn = jnp.take_along_axis(attn, idx, axis=2)
    else:
        table = attn.reshape(_B * _H * _N, _N)
        out_flat = _make_sc_gather()(table, gidx)
        new_attn = out_flat.reshape(_B, _H, _L, _N)
    return new_attn, new_mask, uid


# speedup vs baseline: 2.3955x; 2.3955x over previous
import functools

import jax
import jax.numpy as jnp
from jax import lax
from jax.experimental import pallas as pl
from jax.experimental.pallas import tpu as pltpu
from jax.experimental.pallas import tpu_sc as plsc

_B, _H, _N, _DH = 16, 12, 577, 64
_S = 255
_L = 256
_EPS = 1e-6
_NM1 = _N - 1


def _diag_to_row(col, n):
    bc = jnp.broadcast_to(col, (n, n))
    ii = lax.broadcasted_iota(jnp.int32, (n, n), 0)
    jj = lax.broadcasted_iota(jnp.int32, (n, n), 1)
    return jnp.sum(jnp.where(ii == jj, bc, jnp.zeros_like(bc)), axis=0,
                   keepdims=True)


def _diag_to_col(row, n):
    bc = jnp.broadcast_to(row, (n, n))
    ii = lax.broadcasted_iota(jnp.int32, (n, n), 0)
    jj = lax.broadcasted_iota(jnp.int32, (n, n), 1)
    return jnp.sum(jnp.where(ii == jj, bc, jnp.zeros_like(bc)), axis=1,
                   keepdims=True)


def _cumsum_col(a, n):
    k = 1
    while k < n:
        shifted = jnp.concatenate(
            [jnp.zeros((k, 1), a.dtype), a[: n - k, :]], axis=0)
        a = a + shifted
        k *= 2
    return a


def _part1_body(x_ref, clst_ref, maskt_ref, steps_ref, uid_ref, nm_ref):
    sig = jnp.zeros((_NM1, 1), jnp.float32)
    for h in range(_H):
        xh = x_ref[0, h]
        ssq = jnp.sum(xh * xh, axis=1, keepdims=True)
        norms = jnp.sqrt(ssq[1:, :])
        sig = sig + clst_ref[0, :, h : h + 1] * norms
    denom = jnp.sum(sig) + _EPS
    normed = sig / denom
    cdf = _cumsum_col(normed, _NM1)
    mrow = maskt_ref[0][:, 1:]
    mcol = _diag_to_col(mrow, _NM1)
    cdf = jnp.where(mcol > 0.5, cdf, cdf + 0.1)

    steps = steps_ref[...]
    dist = jnp.abs(steps - cdf)
    dmin = jnp.min(dist, axis=0, keepdims=True)
    nidx = lax.broadcasted_iota(jnp.int32, (_NM1, _S), 0)
    idx_row = jnp.min(jnp.where(dist == dmin, nidx, _NM1), axis=0,
                      keepdims=True)
    ids_row = idx_row + 1

    n = _S
    ii = lax.broadcasted_iota(jnp.int32, (n, n), 0)
    jj = lax.broadcasted_iota(jnp.int32, (n, n), 1)
    ids_col = _diag_to_col(ids_row, n)
    ids_row_bc = jnp.broadcast_to(ids_row, (n, n))
    ids_col_bc = jnp.broadcast_to(ids_col, (n, n))
    lt = (ids_row_bc < ids_col_bc).astype(jnp.int32)
    eq_before = ((ids_row_bc == ids_col_bc) & (jj < ii)).astype(jnp.int32)
    rank_col = jnp.sum(lt + eq_before, axis=1, keepdims=True)

    rank_row = _diag_to_row(rank_col, n)
    rank_row_bc = jnp.broadcast_to(rank_row, (n, n))
    onehot = rank_row_bc == ii
    s_col = jnp.sum(jnp.where(onehot, ids_row_bc, jnp.zeros_like(ids_row_bc)),
                    axis=1, keepdims=True)

    s_prev = jnp.concatenate(
        [jnp.full((1, 1), -1, jnp.int32), s_col[:-1, :]], axis=0)
    keep_col = (s_col != s_prev).astype(jnp.int32)
    keep_row = _diag_to_row(keep_col, n)
    keep_row_bc = jnp.broadcast_to(keep_row, (n, n))
    pos_col = jnp.sum(jnp.where(jj < ii, keep_row_bc, jnp.zeros_like(keep_row_bc)),
                      axis=1, keepdims=True)
    pos_col_bc = jnp.broadcast_to(pos_col, (n, n))
    s_col_bc = jnp.broadcast_to(s_col, (n, n))
    sel = (pos_col_bc == jj) & (keep_col != 0)
    uidv_row = jnp.sum(jnp.where(sel, s_col_bc, jnp.zeros_like(s_col_bc)),
                       axis=0, keepdims=True)

    uid_row = jnp.concatenate(
        [jnp.zeros((1, 1), jnp.int32), uidv_row], axis=1)
    pos0 = lax.broadcasted_iota(jnp.int32, (1, _L), 1)
    nm_row = ((uid_row != 0) | (pos0 == 0)).astype(jnp.int32)

    uid_ref[0] = uid_row
    nm_ref[0] = nm_row


def _part1(x, clst, maskt, steps_row, interpret=False):
    return pl.pallas_call(
        _part1_body,
        grid=(_B,),
        in_specs=[
            pl.BlockSpec((1, _H, _N, _DH), lambda b: (b, 0, 0, 0)),
            pl.BlockSpec((1, _NM1, _H), lambda b: (b, 0, 0)),
            pl.BlockSpec((1, 1, _N), lambda b: (b, 0, 0)),
            pl.BlockSpec((1, _S), lambda b: (0, 0)),
        ],
        out_specs=[
            pl.BlockSpec((1, 1, _L), lambda b: (b, 0, 0)),
            pl.BlockSpec((1, 1, _L), lambda b: (b, 0, 0)),
        ],
        out_shape=[
            jax.ShapeDtypeStruct((_B, 1, _L), jnp.int32),
            jax.ShapeDtypeStruct((_B, 1, _L), jnp.int32),
        ],
        interpret=interpret,
    )(x, clst, maskt.reshape(_B, 1, _N), steps_row)


_DP = 640


def _pad_body(a_ref, o_ref):
    o_ref[:, :, :, : _N] = a_ref[...]
    o_ref[:, :, :, _N:] = jnp.zeros((1, 1, _N, _DP - _N), jnp.float32)


def _pad_table(attn, interpret=False):
    return pl.pallas_call(
        _pad_body,
        grid=(_B, _H),
        in_specs=[pl.BlockSpec((1, 1, _N, _N), lambda b, h: (b, h, 0, 0))],
        out_specs=pl.BlockSpec((1, 1, _N, _DP), lambda b, h: (b, h, 0, 0)),
        out_shape=jax.ShapeDtypeStruct((_B, _H, _N, _DP), jnp.float32),
        interpret=interpret,
    )(attn)


_R = _B * _H * _L
_NW = 32
_RPW = _R // _NW
_CH = 128
_NCH = _RPW // _CH


def _sc_gather_body(table_hbm, gidx_hbm, out_hbm, idx_v, rows_v, sem):
    wid = lax.axis_index("s") * 2 + lax.axis_index("c")
    base = wid * _RPW
    for t in range(_NCH):
        off = base + t * _CH
        pltpu.sync_copy(gidx_hbm.at[pl.ds(off, _CH)], idx_v)
        pltpu.async_copy(table_hbm.at[idx_v], rows_v, sem).wait()
        pltpu.sync_copy(rows_v, out_hbm.at[pl.ds(off, _CH)])


def _make_sc_gather():
    mesh = plsc.VectorSubcoreMesh(core_axis_name="c", subcore_axis_name="s")
    return functools.partial(
        pl.kernel,
        mesh=mesh,
        out_type=jax.ShapeDtypeStruct((_R, _DP), jnp.float32),
        scratch_types=[
            pltpu.VMEM((_CH,), jnp.int32),
            pltpu.VMEM((_CH, _DP), jnp.float32),
            pltpu.SemaphoreType.DMA,
        ],
    )(_sc_gather_body)


def kernel(x, attn, mask, sample_steps):
    clst = jnp.transpose(attn[:, :, 0, 1:], (0, 2, 1))
    maskt = mask.astype(jnp.float32)
    steps_row = sample_steps.reshape(1, _S)
    uid, nm = _part1(x, clst, maskt, steps_row)
    uid = uid.reshape(_B, _L)
    new_mask = nm.reshape(_B, _L).astype(bool)

    boff = (jnp.arange(_B * _H, dtype=jnp.int32) * _N).reshape(_B, _H)
    gidx = (uid[:, None, :] + boff[:, :, None]).reshape(_R)

    table = _pad_table(attn).reshape(_B * _H * _N, _DP)
    out_flat = _make_sc_gather()(table, gidx)
    new_attn = out_flat.reshape(_B, _H, _L, _DP)[:, :, :, : _N]
    return new_attn, new_mask, uid


# speedup vs baseline: 3.3075x; 1.3807x over previous
import functools

import jax
import jax.numpy as jnp
from jax import lax
from jax.experimental import pallas as pl
from jax.experimental.pallas import tpu as pltpu
from jax.experimental.pallas import tpu_sc as plsc

_B, _H, _N, _DH = 16, 12, 577, 64
_S = 255
_L = 256
_EPS = 1e-6
_NM1 = _N - 1


def _diag_to_row(col, n):
    bc = jnp.broadcast_to(col, (n, n))
    ii = lax.broadcasted_iota(jnp.int32, (n, n), 0)
    jj = lax.broadcasted_iota(jnp.int32, (n, n), 1)
    return jnp.sum(jnp.where(ii == jj, bc, jnp.zeros_like(bc)), axis=0,
                   keepdims=True)


def _diag_to_col(row, n):
    bc = jnp.broadcast_to(row, (n, n))
    ii = lax.broadcasted_iota(jnp.int32, (n, n), 0)
    jj = lax.broadcasted_iota(jnp.int32, (n, n), 1)
    return jnp.sum(jnp.where(ii == jj, bc, jnp.zeros_like(bc)), axis=1,
                   keepdims=True)


def _cumsum_col(a, n):
    k = 1
    while k < n:
        shifted = jnp.concatenate(
            [jnp.zeros((k, 1), a.dtype), a[: n - k, :]], axis=0)
        a = a + shifted
        k *= 2
    return a


def _part1_body(x_ref, cls_ref, maskt_ref, steps_ref, uid_ref, nm_ref):
    cols = []
    for h in range(_H):
        xh = x_ref[0, h]
        cols.append(jnp.sum(xh * xh, axis=1, keepdims=True))
    ssq_mat = jnp.concatenate(cols, axis=1)
    ssq_rows = jnp.transpose(ssq_mat, (1, 0))
    norms_rows = jnp.sqrt(ssq_rows[:, 1:])
    sig_row = jnp.zeros((1, _NM1), jnp.float32)
    for h in range(_H):
        sig_row = sig_row + cls_ref[0, h][None, :] * norms_rows[h : h + 1, :]
    sig = jnp.transpose(sig_row, (1, 0))
    denom = jnp.sum(sig) + _EPS
    normed = sig / denom
    cdf = _cumsum_col(normed, _NM1)
    mrow = maskt_ref[0][:, 1:]
    mcol = _diag_to_col(mrow, _NM1)
    cdf = jnp.where(mcol > 0.5, cdf, cdf + 0.1)

    steps = steps_ref[...]
    dist = jnp.abs(steps - cdf)
    dmin = jnp.min(dist, axis=0, keepdims=True)
    nidx = lax.broadcasted_iota(jnp.int32, (_NM1, _S), 0)
    idx_row = jnp.min(jnp.where(dist == dmin, nidx, _NM1), axis=0,
                      keepdims=True)
    ids_row = idx_row + 1

    n = _S
    ii = lax.broadcasted_iota(jnp.int32, (n, n), 0)
    jj = lax.broadcasted_iota(jnp.int32, (n, n), 1)
    ids_col = _diag_to_col(ids_row, n)
    ids_row_bc = jnp.broadcast_to(ids_row, (n, n))
    ids_col_bc = jnp.broadcast_to(ids_col, (n, n))
    lt = (ids_row_bc < ids_col_bc).astype(jnp.int32)
    eq_before = ((ids_row_bc == ids_col_bc) & (jj < ii)).astype(jnp.int32)
    rank_col = jnp.sum(lt + eq_before, axis=1, keepdims=True)

    rank_row = _diag_to_row(rank_col, n)
    rank_row_bc = jnp.broadcast_to(rank_row, (n, n))
    onehot = rank_row_bc == ii
    s_col = jnp.sum(jnp.where(onehot, ids_row_bc, jnp.zeros_like(ids_row_bc)),
                    axis=1, keepdims=True)

    s_prev = jnp.concatenate(
        [jnp.full((1, 1), -1, jnp.int32), s_col[:-1, :]], axis=0)
    keep_col = (s_col != s_prev).astype(jnp.int32)
    keep_row = _diag_to_row(keep_col, n)
    keep_row_bc = jnp.broadcast_to(keep_row, (n, n))
    pos_col = jnp.sum(jnp.where(jj < ii, keep_row_bc, jnp.zeros_like(keep_row_bc)),
                      axis=1, keepdims=True)
    pos_col_bc = jnp.broadcast_to(pos_col, (n, n))
    s_col_bc = jnp.broadcast_to(s_col, (n, n))
    sel = (pos_col_bc == jj) & (keep_col != 0)
    uidv_row = jnp.sum(jnp.where(sel, s_col_bc, jnp.zeros_like(s_col_bc)),
                       axis=0, keepdims=True)

    uid_row = jnp.concatenate(
        [jnp.zeros((1, 1), jnp.int32), uidv_row], axis=1)
    pos0 = lax.broadcasted_iota(jnp.int32, (1, _L), 1)
    nm_row = ((uid_row != 0) | (pos0 == 0)).astype(jnp.int32)

    uid_ref[0] = uid_row
    nm_ref[0] = nm_row


def _part1(x, cls, maskt, steps_row, interpret=False):
    return pl.pallas_call(
        _part1_body,
        grid=(_B,),
        in_specs=[
            pl.BlockSpec((1, _H, _N, _DH), lambda b: (b, 0, 0, 0)),
            pl.BlockSpec((1, _H, _NM1), lambda b: (b, 0, 0)),
            pl.BlockSpec((1, 1, _N), lambda b: (b, 0, 0)),
            pl.BlockSpec((1, _S), lambda b: (0, 0)),
        ],
        out_specs=[
            pl.BlockSpec((1, 1, _L), lambda b: (b, 0, 0)),
            pl.BlockSpec((1, 1, _L), lambda b: (b, 0, 0)),
        ],
        out_shape=[
            jax.ShapeDtypeStruct((_B, 1, _L), jnp.int32),
            jax.ShapeDtypeStruct((_B, 1, _L), jnp.int32),
        ],
        interpret=interpret,
    )(x, cls, maskt.reshape(_B, 1, _N), steps_row)


_DP = 640
_NP = 584


def _pad_body(a_ref, o_ref):
    o_ref[: _N, : _N] = a_ref[0, 0]
    o_ref[: _N, _N:] = jnp.zeros((_N, _DP - _N), jnp.float32)
    o_ref[_N:, :] = jnp.zeros((_NP - _N, _DP), jnp.float32)


def _pad_table(attn, interpret=False):
    return pl.pallas_call(
        _pad_body,
        grid=(_B, _H),
        in_specs=[pl.BlockSpec((1, 1, _N, _N), lambda b, h: (b, h, 0, 0))],
        out_specs=pl.BlockSpec((_NP, _DP), lambda b, h: (b * _H + h, 0)),
        out_shape=jax.ShapeDtypeStruct((_B * _H * _NP, _DP), jnp.float32),
        interpret=interpret,
    )(attn)


_R = _B * _H * _L
_NW = 32
_RPW = _R // _NW
_CH = 128
_NCH = _RPW // _CH


def _sc_gather_body(table_hbm, gidx_hbm, out_hbm, idx_v, rows_v, sem):
    wid = lax.axis_index("s") * 2 + lax.axis_index("c")
    base = wid * _RPW
    for t in range(_NCH):
        off = base + t * _CH
        pltpu.sync_copy(gidx_hbm.at[pl.ds(off, _CH)], idx_v)
        pltpu.async_copy(table_hbm.at[idx_v], rows_v, sem).wait()
        pltpu.sync_copy(rows_v, out_hbm.at[pl.ds(off, _CH)])


def _make_sc_gather():
    mesh = plsc.VectorSubcoreMesh(core_axis_name="c", subcore_axis_name="s")
    return functools.partial(
        pl.kernel,
        mesh=mesh,
        out_type=jax.ShapeDtypeStruct((_R, _DP), jnp.float32),
        scratch_types=[
            pltpu.VMEM((_CH,), jnp.int32),
            pltpu.VMEM((_CH, _DP), jnp.float32),
            pltpu.SemaphoreType.DMA,
        ],
    )(_sc_gather_body)


def kernel(x, attn, mask, sample_steps):
    cls = attn[:, :, 0, 1:]
    maskt = mask.astype(jnp.float32)
    steps_row = sample_steps.reshape(1, _S)
    uid, nm = _part1(x, cls, maskt, steps_row)
    uid = uid.reshape(_B, _L)
    new_mask = nm.reshape(_B, _L).astype(bool)

    boff = (jnp.arange(_B * _H, dtype=jnp.int32) * _NP).reshape(_B, _H)
    gidx = (uid[:, None, :] + boff[:, :, None]).reshape(_R)

    table = _pad_table(attn)
    out_flat = _make_sc_gather()(table, gidx)
    new_attn = out_flat.reshape(_B, _H, _L, _DP)[:, :, :, : _N]
    return new_attn, new_mask, uid


# speedup vs baseline: 3.4751x; 1.0507x over previous
import functools

import jax
import jax.numpy as jnp
from jax import lax
from jax.experimental import pallas as pl
from jax.experimental.pallas import tpu as pltpu
from jax.experimental.pallas import tpu_sc as plsc

_B, _H, _N, _DH = 16, 12, 577, 64
_S = 255
_L = 256
_EPS = 1e-6
_NM1 = _N - 1


def _diag_to_row(col, n):
    bc = jnp.broadcast_to(col, (n, n))
    ii = lax.broadcasted_iota(jnp.int32, (n, n), 0)
    jj = lax.broadcasted_iota(jnp.int32, (n, n), 1)
    return jnp.sum(jnp.where(ii == jj, bc, jnp.zeros_like(bc)), axis=0,
                   keepdims=True)


def _diag_to_col(row, n):
    bc = jnp.broadcast_to(row, (n, n))
    ii = lax.broadcasted_iota(jnp.int32, (n, n), 0)
    jj = lax.broadcasted_iota(jnp.int32, (n, n), 1)
    return jnp.sum(jnp.where(ii == jj, bc, jnp.zeros_like(bc)), axis=1,
                   keepdims=True)


def _cumsum_col(a, n):
    k = 1
    while k < n:
        shifted = jnp.concatenate(
            [jnp.zeros((k, 1), a.dtype), a[: n - k, :]], axis=0)
        a = a + shifted
        k *= 2
    return a


def _part1_body(x_ref, cls_ref, maskt_ref, steps_ref, uid_ref, nm_ref):
    cols = []
    for h in range(_H):
        xh = x_ref[0, h]
        cols.append(jnp.sum(xh * xh, axis=1, keepdims=True))
    ssq_mat = jnp.concatenate(cols, axis=1)
    ssq_rows = jnp.transpose(ssq_mat, (1, 0))
    norms_rows = jnp.sqrt(ssq_rows[:, 1:])
    sig_row = jnp.zeros((1, _NM1), jnp.float32)
    for h in range(_H):
        sig_row = sig_row + cls_ref[0, h][None, :] * norms_rows[h : h + 1, :]
    sig = jnp.transpose(sig_row, (1, 0))
    denom = jnp.sum(sig) + _EPS
    normed = sig / denom
    cdf = _cumsum_col(normed, _NM1)
    mrow = maskt_ref[0][:, 1:]
    mcol = _diag_to_col(mrow, _NM1)
    cdf = jnp.where(mcol > 0.5, cdf, cdf + 0.1)

    steps = steps_ref[...]
    dist = jnp.abs(steps - cdf)
    dmin = jnp.min(dist, axis=0, keepdims=True)
    nidx = lax.broadcasted_iota(jnp.int32, (_NM1, _S), 0)
    idx_row = jnp.min(jnp.where(dist == dmin, nidx, _NM1), axis=0,
                      keepdims=True)
    ids_row = idx_row + 1

    n = _S
    ii = lax.broadcasted_iota(jnp.int32, (n, n), 0)
    jj = lax.broadcasted_iota(jnp.int32, (n, n), 1)
    ids_col = _diag_to_col(ids_row, n)
    ids_row_bc = jnp.broadcast_to(ids_row, (n, n))
    ids_col_bc = jnp.broadcast_to(ids_col, (n, n))
    lt = (ids_row_bc < ids_col_bc).astype(jnp.int32)
    eq_before = ((ids_row_bc == ids_col_bc) & (jj < ii)).astype(jnp.int32)
    rank_col = jnp.sum(lt + eq_before, axis=1, keepdims=True)

    rank_row = _diag_to_row(rank_col, n)
    rank_row_bc = jnp.broadcast_to(rank_row, (n, n))
    onehot = rank_row_bc == ii
    s_col = jnp.sum(jnp.where(onehot, ids_row_bc, jnp.zeros_like(ids_row_bc)),
                    axis=1, keepdims=True)

    s_prev = jnp.concatenate(
        [jnp.full((1, 1), -1, jnp.int32), s_col[:-1, :]], axis=0)
    keep_col = (s_col != s_prev).astype(jnp.int32)
    keep_row = _diag_to_row(keep_col, n)
    keep_row_bc = jnp.broadcast_to(keep_row, (n, n))
    pos_col = jnp.sum(jnp.where(jj < ii, keep_row_bc, jnp.zeros_like(keep_row_bc)),
                      axis=1, keepdims=True)
    pos_col_bc = jnp.broadcast_to(pos_col, (n, n))
    s_col_bc = jnp.broadcast_to(s_col, (n, n))
    sel = (pos_col_bc == jj) & (keep_col != 0)
    uidv_row = jnp.sum(jnp.where(sel, s_col_bc, jnp.zeros_like(s_col_bc)),
                       axis=0, keepdims=True)

    uid_row = jnp.concatenate(
        [jnp.zeros((1, 1), jnp.int32), uidv_row], axis=1)
    pos0 = lax.broadcasted_iota(jnp.int32, (1, _L), 1)
    nm_row = ((uid_row != 0) | (pos0 == 0)).astype(jnp.int32)

    uid_ref[0] = uid_row
    nm_ref[0] = nm_row


def _part1(x, cls, maskt, steps_row, interpret=False):
    return pl.pallas_call(
        _part1_body,
        grid=(_B,),
        in_specs=[
            pl.BlockSpec((1, _H, _N, _DH), lambda b: (b, 0, 0, 0)),
            pl.BlockSpec((1, _H, _NM1), lambda b: (b, 0, 0)),
            pl.BlockSpec((1, 1, _N), lambda b: (b, 0, 0)),
            pl.BlockSpec((1, _S), lambda b: (0, 0)),
        ],
        out_specs=[
            pl.BlockSpec((1, 1, _L), lambda b: (b, 0, 0)),
            pl.BlockSpec((1, 1, _L), lambda b: (b, 0, 0)),
        ],
        out_shape=[
            jax.ShapeDtypeStruct((_B, 1, _L), jnp.int32),
            jax.ShapeDtypeStruct((_B, 1, _L), jnp.int32),
        ],
        interpret=interpret,
    )(x, cls, maskt.reshape(_B, 1, _N), steps_row)


_DP = 640
_NP = 584


_R = _B * _H * _L
_NW = 32
_RPW = _R // _NW
_CH = 64
_NCH = _RPW // _CH


def _sc_gather_body(table_hbm, gidx_hbm, out_hbm,
                    i0, i1, r0, r1, sg0, sg1, so0, so1):
    wid = lax.axis_index("s") * 2 + lax.axis_index("c")
    base = wid * _RPW
    idxs, rows, sgs, sos = (i0, i1), (r0, r1), (sg0, sg1), (so0, so1)
    gh = [None, None]
    oh = [None, None]
    pltpu.sync_copy(gidx_hbm.at[pl.ds(base, _CH)], i0)
    gh[0] = pltpu.async_copy(table_hbm.at[i0], r0, sg0)
    for t in range(_NCH):
        k = t % 2
        kn = (t + 1) % 2
        if t + 1 < _NCH:
            off_n = base + (t + 1) * _CH
            if oh[kn] is not None:
                oh[kn].wait()
            pltpu.sync_copy(gidx_hbm.at[pl.ds(off_n, _CH)], idxs[kn])
            gh[kn] = pltpu.async_copy(table_hbm.at[idxs[kn]], rows[kn], sgs[kn])
        gh[k].wait()
        off = base + t * _CH
        oh[k] = pltpu.async_copy(rows[k], out_hbm.at[pl.ds(off, _CH)], sos[k])
    oh[0].wait()
    oh[1].wait()


def _make_sc_gather():
    mesh = plsc.VectorSubcoreMesh(core_axis_name="c", subcore_axis_name="s")
    return functools.partial(
        pl.kernel,
        mesh=mesh,
        out_type=jax.ShapeDtypeStruct((_R, _DP), jnp.float32),
        scratch_types=[
            pltpu.VMEM((_CH,), jnp.int32),
            pltpu.VMEM((_CH,), jnp.int32),
            pltpu.VMEM((_CH, _DP), jnp.float32),
            pltpu.VMEM((_CH, _DP), jnp.float32),
            pltpu.SemaphoreType.DMA,
            pltpu.SemaphoreType.DMA,
            pltpu.SemaphoreType.DMA,
            pltpu.SemaphoreType.DMA,
        ],
    )(_sc_gather_body)


def kernel(x, attn, mask, sample_steps):
    cls = attn[:, :, 0, 1:]
    maskt = mask.astype(jnp.float32)
    steps_row = sample_steps.reshape(1, _S)
    uid, nm = _part1(x, cls, maskt, steps_row)
    uid = uid.reshape(_B, _L)
    new_mask = nm.reshape(_B, _L).astype(bool)

    boff = (jnp.arange(_B * _H, dtype=jnp.int32) * _NP).reshape(_B, _H)
    gidx = (uid[:, None, :] + boff[:, :, None]).reshape(_R)

    table = jnp.pad(attn, ((0, 0), (0, 0), (0, _NP - _N), (0, _DP - _N))
                    ).reshape(_B * _H * _NP, _DP)
    out_flat = _make_sc_gather()(table, gidx)
    new_attn = out_flat.reshape(_B, _H, _L, _DP)[:, :, :, : _N]
    return new_attn, new_mask, uid


# speedup vs baseline: 3.6781x; 1.0584x over previous
import functools

import jax
import jax.numpy as jnp
from jax import lax
from jax.experimental import pallas as pl
from jax.experimental.pallas import tpu as pltpu
from jax.experimental.pallas import tpu_sc as plsc

_B, _H, _N, _DH = 16, 12, 577, 64
_S = 255
_L = 256
_EPS = 1e-6
_NM1 = _N - 1


def _diag_to_row(col, n):
    bc = jnp.broadcast_to(col, (n, n))
    ii = lax.broadcasted_iota(jnp.int32, (n, n), 0)
    jj = lax.broadcasted_iota(jnp.int32, (n, n), 1)
    return jnp.sum(jnp.where(ii == jj, bc, jnp.zeros_like(bc)), axis=0,
                   keepdims=True)


def _diag_to_col(row, n):
    bc = jnp.broadcast_to(row, (n, n))
    ii = lax.broadcasted_iota(jnp.int32, (n, n), 0)
    jj = lax.broadcasted_iota(jnp.int32, (n, n), 1)
    return jnp.sum(jnp.where(ii == jj, bc, jnp.zeros_like(bc)), axis=1,
                   keepdims=True)


def _cumsum_col(a, n):
    k = 1
    while k < n:
        shifted = jnp.concatenate(
            [jnp.zeros((k, 1), a.dtype), a[: n - k, :]], axis=0)
        a = a + shifted
        k *= 2
    return a


def _part1_body(x_ref, cls_ref, maskt_ref, steps_ref, uid_ref, nm_ref):
    rows = []
    for h in range(_H):
        xh = x_ref[0, h]
        rows.append(jnp.sum(xh * xh, axis=0, keepdims=True))
    ssq_rows = jnp.concatenate(rows, axis=0)
    norms_rows = jnp.sqrt(ssq_rows[:, 1:])
    sig_row = jnp.zeros((1, _NM1), jnp.float32)
    for h in range(_H):
        sig_row = sig_row + cls_ref[0, h][None, :] * norms_rows[h : h + 1, :]
    sig = jnp.transpose(sig_row, (1, 0))
    denom = jnp.sum(sig) + _EPS
    normed = sig / denom
    cdf = _cumsum_col(normed, _NM1)
    mrow = maskt_ref[0][:, 1:]
    mcol = _diag_to_col(mrow, _NM1)
    cdf = jnp.where(mcol > 0.5, cdf, cdf + 0.1)

    steps = steps_ref[...]
    dist = jnp.abs(steps - cdf)
    dmin = jnp.min(dist, axis=0, keepdims=True)
    nidx = lax.broadcasted_iota(jnp.int32, (_NM1, _S), 0)
    idx_row = jnp.min(jnp.where(dist == dmin, nidx, _NM1), axis=0,
                      keepdims=True)
    ids_row = idx_row + 1

    n = _S
    ii = lax.broadcasted_iota(jnp.int32, (n, n), 0)
    jj = lax.broadcasted_iota(jnp.int32, (n, n), 1)
    ids_col = _diag_to_col(ids_row, n)
    ids_row_bc = jnp.broadcast_to(ids_row, (n, n))
    ids_col_bc = jnp.broadcast_to(ids_col, (n, n))
    lt = (ids_row_bc < ids_col_bc).astype(jnp.int32)
    eq_before = ((ids_row_bc == ids_col_bc) & (jj < ii)).astype(jnp.int32)
    rank_col = jnp.sum(lt + eq_before, axis=1, keepdims=True)

    rank_row = _diag_to_row(rank_col, n)
    rank_row_bc = jnp.broadcast_to(rank_row, (n, n))
    onehot = rank_row_bc == ii
    s_col = jnp.sum(jnp.where(onehot, ids_row_bc, jnp.zeros_like(ids_row_bc)),
                    axis=1, keepdims=True)

    s_prev = jnp.concatenate(
        [jnp.full((1, 1), -1, jnp.int32), s_col[:-1, :]], axis=0)
    keep_col = (s_col != s_prev).astype(jnp.int32)
    keep_row = _diag_to_row(keep_col, n)
    keep_row_bc = jnp.broadcast_to(keep_row, (n, n))
    pos_col = jnp.sum(jnp.where(jj < ii, keep_row_bc, jnp.zeros_like(keep_row_bc)),
                      axis=1, keepdims=True)
    pos_col_bc = jnp.broadcast_to(pos_col, (n, n))
    s_col_bc = jnp.broadcast_to(s_col, (n, n))
    sel = (pos_col_bc == jj) & (keep_col != 0)
    uidv_row = jnp.sum(jnp.where(sel, s_col_bc, jnp.zeros_like(s_col_bc)),
                       axis=0, keepdims=True)

    uid_row = jnp.concatenate(
        [jnp.zeros((1, 1), jnp.int32), uidv_row], axis=1)
    pos0 = lax.broadcasted_iota(jnp.int32, (1, _L), 1)
    nm_row = ((uid_row != 0) | (pos0 == 0)).astype(jnp.int32)

    uid_ref[0] = uid_row
    nm_ref[0] = nm_row


def _part1(x, cls, maskt, steps_row, interpret=False):
    return pl.pallas_call(
        _part1_body,
        grid=(_B,),
        in_specs=[
            pl.BlockSpec((1, _H, _DH, _N), lambda b: (b, 0, 0, 0)),
            pl.BlockSpec((1, _H, _NM1), lambda b: (b, 0, 0)),
            pl.BlockSpec((1, 1, _N), lambda b: (b, 0, 0)),
            pl.BlockSpec((1, _S), lambda b: (0, 0)),
        ],
        out_specs=[
            pl.BlockSpec((1, 1, _L), lambda b: (b, 0, 0)),
            pl.BlockSpec((1, 1, _L), lambda b: (b, 0, 0)),
        ],
        out_shape=[
            jax.ShapeDtypeStruct((_B, 1, _L), jnp.int32),
            jax.ShapeDtypeStruct((_B, 1, _L), jnp.int32),
        ],
        interpret=interpret,
    )(x, cls, maskt.reshape(_B, 1, _N), steps_row)


_DP = 640
_NP = 584


_R = _B * _H * _L
_NW = 32
_RPW = _R // _NW
_CH = 64
_NCH = _RPW // _CH


def _sc_gather_body(table_hbm, gidx_hbm, out_hbm,
                    i0, i1, r0, r1, sg0, sg1, so0, so1):
    wid = lax.axis_index("s") * 2 + lax.axis_index("c")
    base = wid * _RPW
    idxs, rows, sgs, sos = (i0, i1), (r0, r1), (sg0, sg1), (so0, so1)
    gh = [None, None]
    oh = [None, None]
    pltpu.sync_copy(gidx_hbm.at[pl.ds(base, _CH)], i0)
    gh[0] = pltpu.async_copy(table_hbm.at[i0], r0, sg0)
    for t in range(_NCH):
        k = t % 2
        kn = (t + 1) % 2
        if t + 1 < _NCH:
            off_n = base + (t + 1) * _CH
            if oh[kn] is not None:
                oh[kn].wait()
            pltpu.sync_copy(gidx_hbm.at[pl.ds(off_n, _CH)], idxs[kn])
            gh[kn] = pltpu.async_copy(table_hbm.at[idxs[kn]], rows[kn], sgs[kn])
        gh[k].wait()
        off = base + t * _CH
        oh[k] = pltpu.async_copy(rows[k], out_hbm.at[pl.ds(off, _CH)], sos[k])
    oh[0].wait()
    oh[1].wait()


def _make_sc_gather():
    mesh = plsc.VectorSubcoreMesh(core_axis_name="c", subcore_axis_name="s")
    return functools.partial(
        pl.kernel,
        mesh=mesh,
        out_type=jax.ShapeDtypeStruct((_R, _DP), jnp.float32),
        scratch_types=[
            pltpu.VMEM((_CH,), jnp.int32),
            pltpu.VMEM((_CH,), jnp.int32),
            pltpu.VMEM((_CH, _DP), jnp.float32),
            pltpu.VMEM((_CH, _DP), jnp.float32),
            pltpu.SemaphoreType.DMA,
            pltpu.SemaphoreType.DMA,
            pltpu.SemaphoreType.DMA,
            pltpu.SemaphoreType.DMA,
        ],
    )(_sc_gather_body)


def kernel(x, attn, mask, sample_steps):
    xt = jnp.transpose(x, (0, 1, 3, 2))
    cls = attn[:, :, 0, 1:]
    maskt = mask.astype(jnp.float32)
    steps_row = sample_steps.reshape(1, _S)
    uid, nm = _part1(xt, cls, maskt, steps_row)
    uid = uid.reshape(_B, _L)
    new_mask = nm.reshape(_B, _L).astype(bool)

    boff = (jnp.arange(_B * _H, dtype=jnp.int32) * _NP).reshape(_B, _H)
    gidx = (uid[:, None, :] + boff[:, :, None]).reshape(_R)

    table = jnp.pad(attn, ((0, 0), (0, 0), (0, _NP - _N), (0, _DP - _N))
                    ).reshape(_B * _H * _NP, _DP)
    out_flat = _make_sc_gather()(table, gidx)
    new_attn = out_flat.reshape(_B, _H, _L, _DP)[:, :, :, : _N]
    return new_attn, new_mask, uid


# speedup vs baseline: 5.6066x; 1.5243x over previous
import functools

import jax
import jax.numpy as jnp
from jax import lax
from jax.experimental import pallas as pl
from jax.experimental.pallas import tpu as pltpu
from jax.experimental.pallas import tpu_sc as plsc

_B, _H, _N, _DH = 16, 12, 577, 64
_S = 255
_L = 256
_EPS = 1e-6
_NM1 = _N - 1


def _diag_to_row(col, n):
    bc = jnp.broadcast_to(col, (n, n))
    ii = lax.broadcasted_iota(jnp.int32, (n, n), 0)
    jj = lax.broadcasted_iota(jnp.int32, (n, n), 1)
    return jnp.sum(jnp.where(ii == jj, bc, jnp.zeros_like(bc)), axis=0,
                   keepdims=True)


def _diag_to_col(row, n):
    bc = jnp.broadcast_to(row, (n, n))
    ii = lax.broadcasted_iota(jnp.int32, (n, n), 0)
    jj = lax.broadcasted_iota(jnp.int32, (n, n), 1)
    return jnp.sum(jnp.where(ii == jj, bc, jnp.zeros_like(bc)), axis=1,
                   keepdims=True)


def _cumsum_col(a, n):
    k = 1
    while k < n:
        shifted = jnp.concatenate(
            [jnp.zeros((k, 1), a.dtype), a[: n - k, :]], axis=0)
        a = a + shifted
        k *= 2
    return a


def _part1_body(x_ref, cls_ref, maskt_ref, steps_ref, uid_ref, nm_ref):
    rows = []
    for h in range(_H):
        xh = x_ref[0, h]
        rows.append(jnp.sum(xh * xh, axis=0, keepdims=True))
    ssq_rows = jnp.concatenate(rows, axis=0)
    norms_rows = jnp.sqrt(ssq_rows[:, 1:])
    sig_row = jnp.zeros((1, _NM1), jnp.float32)
    for h in range(_H):
        sig_row = sig_row + cls_ref[0, h][None, :] * norms_rows[h : h + 1, :]
    sig = jnp.transpose(sig_row, (1, 0))
    denom = jnp.sum(sig) + _EPS
    normed = sig / denom
    cdf = _cumsum_col(normed, _NM1)
    mrow = maskt_ref[0][:, 1:]
    mcol = _diag_to_col(mrow, _NM1)
    cdf = jnp.where(mcol > 0.5, cdf, cdf + 0.1)

    steps = steps_ref[...]
    dist = jnp.abs(steps - cdf)
    dmin = jnp.min(dist, axis=0, keepdims=True)
    nidx = lax.broadcasted_iota(jnp.int32, (_NM1, _S), 0)
    idx_row = jnp.min(jnp.where(dist == dmin, nidx, _NM1), axis=0,
                      keepdims=True)
    ids_row = idx_row + 1

    n = _S
    ii = lax.broadcasted_iota(jnp.int32, (n, n), 0)
    jj = lax.broadcasted_iota(jnp.int32, (n, n), 1)
    ids_col = _diag_to_col(ids_row, n)
    ids_row_bc = jnp.broadcast_to(ids_row, (n, n))
    ids_col_bc = jnp.broadcast_to(ids_col, (n, n))
    lt = (ids_row_bc < ids_col_bc).astype(jnp.int32)
    eq_before = ((ids_row_bc == ids_col_bc) & (jj < ii)).astype(jnp.int32)
    rank_col = jnp.sum(lt + eq_before, axis=1, keepdims=True)

    rank_row = _diag_to_row(rank_col, n)
    rank_row_bc = jnp.broadcast_to(rank_row, (n, n))
    onehot = rank_row_bc == ii
    s_col = jnp.sum(jnp.where(onehot, ids_row_bc, jnp.zeros_like(ids_row_bc)),
                    axis=1, keepdims=True)

    s_prev = jnp.concatenate(
        [jnp.full((1, 1), -1, jnp.int32), s_col[:-1, :]], axis=0)
    keep_col = (s_col != s_prev).astype(jnp.int32)
    keep_row = _diag_to_row(keep_col, n)
    keep_row_bc = jnp.broadcast_to(keep_row, (n, n))
    pos_col = jnp.sum(jnp.where(jj < ii, keep_row_bc, jnp.zeros_like(keep_row_bc)),
                      axis=1, keepdims=True)
    pos_col_bc = jnp.broadcast_to(pos_col, (n, n))
    s_col_bc = jnp.broadcast_to(s_col, (n, n))
    sel = (pos_col_bc == jj) & (keep_col != 0)
    uidv_row = jnp.sum(jnp.where(sel, s_col_bc, jnp.zeros_like(s_col_bc)),
                       axis=0, keepdims=True)

    uid_row = jnp.concatenate(
        [jnp.zeros((1, 1), jnp.int32), uidv_row], axis=1)
    pos0 = lax.broadcasted_iota(jnp.int32, (1, _L), 1)
    nm_row = ((uid_row != 0) | (pos0 == 0)).astype(jnp.int32)

    uid_ref[0] = uid_row
    nm_ref[0] = nm_row


def _part1(x, cls, maskt, steps_row, interpret=False):
    return pl.pallas_call(
        _part1_body,
        grid=(_B,),
        in_specs=[
            pl.BlockSpec((1, _H, _DH, _N), lambda b: (b, 0, 0, 0)),
            pl.BlockSpec((1, _H, _NM1), lambda b: (b, 0, 0)),
            pl.BlockSpec((1, 1, _N), lambda b: (b, 0, 0)),
            pl.BlockSpec((1, _S), lambda b: (0, 0)),
        ],
        out_specs=[
            pl.BlockSpec((1, 1, _L), lambda b: (b, 0, 0)),
            pl.BlockSpec((1, 1, _L), lambda b: (b, 0, 0)),
        ],
        out_shape=[
            jax.ShapeDtypeStruct((_B, 1, _L), jnp.int32),
            jax.ShapeDtypeStruct((_B, 1, _L), jnp.int32),
        ],
        interpret=interpret,
    )(x, cls, maskt.reshape(_B, 1, _N), steps_row)


_DP = 640
_NP = 584
_BG = 8


def _pad_body(a_ref, o_ref):
    for bb in range(_BG):
        o_ref[bb, : _N, : _N] = a_ref[0, :, bb, :]


def _pad_table(attn_v, interpret=False):
    return pl.pallas_call(
        _pad_body,
        grid=(_H, _B // _BG),
        in_specs=[pl.BlockSpec((1, _N, _BG, _N), lambda h, g: (h, 0, g, 0))],
        out_specs=pl.BlockSpec((_BG, _NP, _DP), lambda h, g: (h * (_B // _BG) + g, 0, 0)),
        out_shape=jax.ShapeDtypeStruct((_B * _H, _NP, _DP), jnp.float32),
        interpret=interpret,
    )(attn_v)


_R = _B * _H * _L
_NW = 32
_RPW = _R // _NW
_CH = 64
_NCH = _RPW // _CH


def _sc_gather_body(table_hbm, gidx_hbm, out_hbm,
                    i0, i1, r0, r1, sg0, sg1, so0, so1):
    wid = lax.axis_index("s") * 2 + lax.axis_index("c")
    base = wid * _RPW
    idxs, rows, sgs, sos = (i0, i1), (r0, r1), (sg0, sg1), (so0, so1)
    gh = [None, None]
    oh = [None, None]
    pltpu.sync_copy(gidx_hbm.at[pl.ds(base, _CH)], i0)
    gh[0] = pltpu.async_copy(table_hbm.at[i0], r0, sg0)
    for t in range(_NCH):
        k = t % 2
        kn = (t + 1) % 2
        if t + 1 < _NCH:
            off_n = base + (t + 1) * _CH
            if oh[kn] is not None:
                oh[kn].wait()
            pltpu.sync_copy(gidx_hbm.at[pl.ds(off_n, _CH)], idxs[kn])
            gh[kn] = pltpu.async_copy(table_hbm.at[idxs[kn]], rows[kn], sgs[kn])
        gh[k].wait()
        off = base + t * _CH
        oh[k] = pltpu.async_copy(rows[k], out_hbm.at[pl.ds(off, _CH)], sos[k])
    oh[0].wait()
    oh[1].wait()


def _make_sc_gather():
    mesh = plsc.VectorSubcoreMesh(core_axis_name="c", subcore_axis_name="s")
    return functools.partial(
        pl.kernel,
        mesh=mesh,
        out_type=jax.ShapeDtypeStruct((_R, _DP), jnp.float32),
        scratch_types=[
            pltpu.VMEM((_CH,), jnp.int32),
            pltpu.VMEM((_CH,), jnp.int32),
            pltpu.VMEM((_CH, _DP), jnp.float32),
            pltpu.VMEM((_CH, _DP), jnp.float32),
            pltpu.SemaphoreType.DMA,
            pltpu.SemaphoreType.DMA,
            pltpu.SemaphoreType.DMA,
            pltpu.SemaphoreType.DMA,
        ],
    )(_sc_gather_body)


def kernel(x, attn, mask, sample_steps):
    xt = jnp.transpose(x, (0, 1, 3, 2))
    cls = attn[:, :, 0, 1:]
    maskt = mask.astype(jnp.float32)
    steps_row = sample_steps.reshape(1, _S)
    uid, nm = _part1(xt, cls, maskt, steps_row)
    uid = uid.reshape(_B, _L)
    new_mask = nm.reshape(_B, _L).astype(bool)

    hh = jnp.arange(_H, dtype=jnp.int32)[None, :]
    bb2 = jnp.arange(_B, dtype=jnp.int32)[:, None]
    boff = (hh * _B + bb2) * _NP
    gidx = (uid[:, None, :] + boff[:, :, None]).reshape(_R)

    attn_v = jnp.transpose(attn, (1, 2, 0, 3))
    table = _pad_table(attn_v).reshape(_B * _H * _NP, _DP)
    out_flat = _make_sc_gather()(table, gidx)
    new_attn = out_flat.reshape(_B, _H, _L, _DP)[:, :, :, : _N]
    return new_attn, new_mask, uid
